# baseline (speedup 1.0000x reference)
import functools

import jax
import jax.numpy as jnp
from jax import lax
from jax.experimental import pallas as pl
from jax.experimental.pallas import tpu as pltpu
from jax.experimental.pallas import tpu_sc as plsc

NSK = 100
E = 64
NT = 225
FL = 3
MC = 4
B, L = 8, 100
NTOK = B * L
PER = 176
ZERO = 175
NW = 32
TPW = NTOK // NW
TPAD = 32
SLOTS = 7 * MC
SK_BLK = 10
_STAGE = (NSK * PER) // 16


def _t_body(wp_ref, emb_ref, wc_ref, c_ref, r_ref, p_ref, a_ref,
            out_ref, meta_ref):
    wc = wc_ref[...]
    for ss in range(SK_BLK):
        aa = wp_ref[:, 0, ss, :]
        b1 = aa[:, :E] * emb_ref[0, pl.ds(ss, 1), :]
        temb = lax.dot_general(
            wc, b1, (((0,), (1,)), ((), ())),
            preferred_element_type=jnp.float32)
        tfull = temb + aa[:, E:].T
        top = tfull[0:25] + tfull[25:50] + tfull[50:75]
        out_ref[ss] = jnp.concatenate(
            [top, tfull[75:225], jnp.zeros((1, E), jnp.float32)], axis=0)

    @pl.when(pl.program_id(0) == 0)
    def _():
        zpad = jnp.zeros((NW, TPAD - TPW), jnp.int32)
        for f in range(MC):
            meta_ref[:, f, :] = jnp.concatenate([c_ref[:, :, f], zpad], axis=1)
            meta_ref[:, MC + f, :] = jnp.concatenate([r_ref[:, :, f], zpad], axis=1)
        for k in range(MC * FL):
            meta_ref[:, 2 * MC + k, :] = jnp.concatenate([p_ref[:, :, k], zpad], axis=1)
            meta_ref[:, 2 * MC + MC * FL + k, :] = jnp.concatenate(
                [a_ref[:, :, k], zpad], axis=1)


_t_precompute = pl.pallas_call(
    _t_body,
    grid=(NSK // SK_BLK,),
    in_specs=[
        pl.BlockSpec((E, 1, SK_BLK, E + NT), lambda i: (0, i, 0, 0)),
        pl.BlockSpec((1, SK_BLK, E), lambda i: (i, 0, 0)),
        pl.BlockSpec((E, NT), lambda i: (0, 0)),
        pl.BlockSpec((NW, TPW, MC), lambda i: (0, 0, 0)),
        pl.BlockSpec((NW, TPW, MC), lambda i: (0, 0, 0)),
        pl.BlockSpec((NW, TPW, MC * FL), lambda i: (0, 0, 0)),
        pl.BlockSpec((NW, TPW, MC * FL), lambda i: (0, 0, 0)),
    ],
    out_specs=[
        pl.BlockSpec((SK_BLK, PER, E), lambda i: (i, 0, 0)),
        pl.BlockSpec((NW, 2 * MC + 2 * MC * FL, TPAD), lambda i: (0, 0, 0)),
    ],
    out_shape=[
        jax.ShapeDtypeStruct((NSK, PER, E), jnp.float32),
        jax.ShapeDtypeStruct((NW, 2 * MC + 2 * MC * FL, TPAD), jnp.int32),
    ],
)


def _sc_body(meta_hbm, t_hbm, bias_hbm, out_hbm,
             meta_v, idx_v, bias_v, out_v, shared, sem, sem2):
    sid = lax.axis_index("s")
    wid = sid * 2 + lax.axis_index("c")

    stage = pltpu.async_copy(t_hbm.at[pl.ds(sid * _STAGE, _STAGE)],
                             shared.at[pl.ds(sid * _STAGE, _STAGE)], sem2)

    pltpu.sync_copy(meta_hbm.at[wid], meta_v)
    pltpu.sync_copy(bias_hbm, bias_v)

    for chunk in range(2):
        sl = pl.ds(chunk * 16, 16)
        cs = [meta_v[f, sl] for f in range(MC)]
        rs = [meta_v[MC + f, sl] for f in range(MC)]
        ps = [meta_v[2 * MC + k, sl] for k in range(MC * FL)]
        asv = [meta_v[2 * MC + MC * FL + k, sl] for k in range(MC * FL)]
        j = 0
        for f in range(MC):
            win = None
            for g in range(f + 1, MC):
                neq = cs[f] != cs[g]
                win = neq if win is None else win & neq
            base = cs[f] * PER
            seven = [base + rs[f]]
            for q in range(FL):
                seven.append(base + 25 + 25 * q + ps[FL * f + q])
            for q in range(FL):
                seven.append(base + 100 + 25 * q + asv[FL * f + q])
            for vec in seven:
                v = vec if win is None else jnp.where(win, vec, ZERO)
                idx_v[j, pl.ds(chunk * 16, 16)] = v
                j += 1

    bias_regs = [bias_v[pl.ds(k * 16, 16)] for k in range(E // 16)]

    def initb(t, carry):
        for k in range(E // 16):
            out_v[t, pl.ds(k * 16, 16)] = bias_regs[k]
        return carry

    lax.fori_loop(0, TPAD, initb, 0)

    stage.wait()
    plsc.subcore_barrier()

    copies = [
        pltpu.async_copy(shared.at[idx_v.at[j]], out_v, sem, add=True)
        for j in range(SLOTS)
    ]
    for cp in copies:
        cp.wait()
    pltpu.sync_copy(out_v.at[pl.ds(0, TPW)], out_hbm.at[pl.ds(wid * TPW, TPW)])


@functools.cache
def _sc_gather_fn():
    return functools.partial(
        pl.kernel,
        out_type=jax.ShapeDtypeStruct((NTOK, E), jnp.float32),
        mesh=plsc.VectorSubcoreMesh(
            core_axis_name="c", subcore_axis_name="s", num_cores=2, num_subcores=16),
        scratch_types=[
            pltpu.VMEM((2 * MC + 2 * MC * FL, TPAD), jnp.int32),
            pltpu.VMEM((SLOTS, TPAD), jnp.int32),
            pltpu.VMEM((E,), jnp.float32),
            pltpu.VMEM((TPAD, E), jnp.float32),
            pltpu.VMEM_SHARED((NSK * PER, E), jnp.float32),
            pltpu.SemaphoreType.DMA,
            pltpu.SemaphoreType.DMA,
        ],
        compiler_params=pltpu.CompilerParams(use_tc_tiling_on_sc=False),
    )(_sc_body)


@jax.jit
def kernel(concepts, rgaps, pcounts, acounts, emb_table_skill, W_cemb, W_pre, b_pre):
    c3 = concepts.reshape(NW, TPW, MC).astype(jnp.int32)
    r3 = rgaps.reshape(NW, TPW, MC).astype(jnp.int32)
    p3 = pcounts.reshape(NW, TPW, MC * FL).astype(jnp.int32)
    a3 = acounts.reshape(NW, TPW, MC * FL).astype(jnp.int32)

    wp4 = W_pre.reshape(E, NSK // SK_BLK, SK_BLK, E + NT)
    emb3 = emb_table_skill.reshape(NSK // SK_BLK, SK_BLK, E)
    table, meta = _t_precompute(wp4, emb3, W_cemb, c3, r3, p3, a3)

    out = _sc_gather_fn()(meta, table.reshape(NSK * PER, E), b_pre)
    return out.reshape(B, L, E)


# speedup vs baseline: 1.1394x; 1.1394x over previous
import functools

import jax
import jax.numpy as jnp
from jax import lax
from jax.experimental import pallas as pl
from jax.experimental.pallas import tpu as pltpu
from jax.experimental.pallas import tpu_sc as plsc

NSK = 100
E = 64
NT = 225
FL = 3
MC = 4
B, L = 8, 100
NTOK = B * L
PER = 176
ZERO = 175
NW = 32
TPW = NTOK // NW
TPAD = 32
SLOTS = 7 * MC
SK_BLK = 10
_STAGE = (NSK * PER) // 16


def _t_body(wp_ref, emb_ref, wc_ref, out_ref):
    wc = wc_ref[...]
    for ss in range(SK_BLK):
        aa = wp_ref[:, 0, ss, :]
        b1 = aa[:, :E] * emb_ref[0, pl.ds(ss, 1), :]
        temb = lax.dot_general(
            wc, b1, (((0,), (1,)), ((), ())),
            preferred_element_type=jnp.float32)
        tfull = temb + aa[:, E:].T
        top = tfull[0:25] + tfull[25:50] + tfull[50:75]
        out_ref[ss] = jnp.concatenate(
            [top, tfull[75:225], jnp.zeros((1, E), jnp.float32)], axis=0)


_t_precompute = pl.pallas_call(
    _t_body,
    grid=(NSK // SK_BLK,),
    in_specs=[
        pl.BlockSpec((E, 1, SK_BLK, E + NT), lambda i: (0, i, 0, 0)),
        pl.BlockSpec((1, SK_BLK, E), lambda i: (i, 0, 0)),
        pl.BlockSpec((E, NT), lambda i: (0, 0)),
    ],
    out_specs=pl.BlockSpec((SK_BLK, PER, E), lambda i: (i, 0, 0)),
    out_shape=jax.ShapeDtypeStruct((NSK, PER, E), jnp.float32),
)


def _sc_body(meta_hbm, t_hbm, bias_hbm, out_hbm,
             meta_v, idx_v, bias_v, out_v, shared, sem, sem2):
    sid = lax.axis_index("s")
    wid = sid * 2 + lax.axis_index("c")

    stage = pltpu.async_copy(t_hbm.at[pl.ds(sid * _STAGE, _STAGE)],
                             shared.at[pl.ds(sid * _STAGE, _STAGE)], sem2)

    pltpu.sync_copy(meta_hbm.at[wid], meta_v)
    pltpu.sync_copy(bias_hbm, bias_v)

    for chunk in range(2):
        sl = pl.ds(chunk * 16, 16)
        cs = [meta_v[f, sl] for f in range(MC)]
        rs = [meta_v[MC + f, sl] for f in range(MC)]
        ps = [meta_v[2 * MC + k, sl] for k in range(MC * FL)]
        asv = [meta_v[2 * MC + MC * FL + k, sl] for k in range(MC * FL)]
        j = 0
        for f in range(MC):
            win = None
            for g in range(f + 1, MC):
                neq = cs[f] != cs[g]
                win = neq if win is None else win & neq
            base = cs[f] * PER
            seven = [base + rs[f]]
            for q in range(FL):
                seven.append(base + 25 + 25 * q + ps[FL * f + q])
            for q in range(FL):
                seven.append(base + 100 + 25 * q + asv[FL * f + q])
            for vec in seven:
                v = vec if win is None else jnp.where(win, vec, ZERO)
                idx_v[j, pl.ds(chunk * 16, 16)] = v
                j += 1

    bias_regs = [bias_v[pl.ds(k * 16, 16)] for k in range(E // 16)]

    def initb(t, carry):
        for k in range(E // 16):
            out_v[t, pl.ds(k * 16, 16)] = bias_regs[k]
        return carry

    lax.fori_loop(0, TPAD, initb, 0)

    stage.wait()
    plsc.subcore_barrier()

    copies = [
        pltpu.async_copy(shared.at[idx_v.at[j]], out_v, sem, add=True)
        for j in range(SLOTS)
    ]
    for cp in copies:
        cp.wait()
    pltpu.sync_copy(out_v.at[pl.ds(0, TPW)], out_hbm.at[pl.ds(wid * TPW, TPW)])


@functools.cache
def _sc_gather_fn():
    return functools.partial(
        pl.kernel,
        out_type=jax.ShapeDtypeStruct((NTOK, E), jnp.float32),
        mesh=plsc.VectorSubcoreMesh(
            core_axis_name="c", subcore_axis_name="s", num_cores=2, num_subcores=16),
        scratch_types=[
            pltpu.VMEM((2 * MC + 2 * MC * FL, TPAD), jnp.int32),
            pltpu.VMEM((SLOTS, TPAD), jnp.int32),
            pltpu.VMEM((E,), jnp.float32),
            pltpu.VMEM((TPAD, E), jnp.float32),
            pltpu.VMEM_SHARED((NSK * PER, E), jnp.float32),
            pltpu.SemaphoreType.DMA,
            pltpu.SemaphoreType.DMA,
        ],
        compiler_params=pltpu.CompilerParams(use_tc_tiling_on_sc=False),
    )(_sc_body)


@jax.jit
def kernel(concepts, rgaps, pcounts, acounts, emb_table_skill, W_cemb, W_pre, b_pre):
    cw = concepts.reshape(NW, TPW, MC).astype(jnp.int32).transpose(0, 2, 1)
    rw = rgaps.reshape(NW, TPW, MC).astype(jnp.int32).transpose(0, 2, 1)
    pw = pcounts.reshape(NW, TPW, MC * FL).astype(jnp.int32).transpose(0, 2, 1)
    aw = acounts.reshape(NW, TPW, MC * FL).astype(jnp.int32).transpose(0, 2, 1)
    meta = jnp.concatenate([cw, rw, pw, aw], axis=1)
    meta = jnp.pad(meta, ((0, 0), (0, 0), (0, TPAD - TPW)))

    wp4 = W_pre.reshape(E, NSK // SK_BLK, SK_BLK, E + NT)
    emb3 = emb_table_skill.reshape(NSK // SK_BLK, SK_BLK, E)
    table = _t_precompute(wp4, emb3, W_cemb)

    out = _sc_gather_fn()(meta, table.reshape(NSK * PER, E), b_pre)
    return out.reshape(B, L, E)


# speedup vs baseline: 1.2198x; 1.0706x over previous
import functools

import jax
import jax.numpy as jnp
from jax import lax
from jax.experimental import pallas as pl
from jax.experimental.pallas import tpu as pltpu
from jax.experimental.pallas import tpu_sc as plsc

NSK = 100
E = 64
NT = 225
FL = 3
MC = 4
B, L = 8, 100
NTOK = B * L
PER = 176
ZERO = 175
NW = 32
TPW = NTOK // NW
TPAD = 32
SLOTS = 7 * MC
SK_BLK = 20
_STAGE = (NSK * PER) // 16


def _t_body(wp_ref, emb_ref, wc_ref, out_ref):
    wc = wc_ref[...]
    for ss in range(SK_BLK):
        aa = wp_ref[:, 0, ss, :]
        b1 = aa[:, :E] * emb_ref[0, pl.ds(ss, 1), :]
        temb = lax.dot_general(
            wc, b1, (((0,), (1,)), ((), ())),
            preferred_element_type=jnp.float32)
        tfull = temb + aa[:, E:].T
        top = tfull[0:25] + tfull[25:50] + tfull[50:75]
        out_ref[ss] = jnp.concatenate(
            [top, tfull[75:225], jnp.zeros((1, E), jnp.float32)], axis=0)


_t_precompute = pl.pallas_call(
    _t_body,
    grid=(NSK // SK_BLK,),
    in_specs=[
        pl.BlockSpec((E, 1, SK_BLK, E + NT), lambda i: (0, i, 0, 0)),
        pl.BlockSpec((1, SK_BLK, E), lambda i: (i, 0, 0)),
        pl.BlockSpec((E, NT), lambda i: (0, 0)),
    ],
    out_specs=pl.BlockSpec((SK_BLK, PER, E), lambda i: (i, 0, 0)),
    out_shape=jax.ShapeDtypeStruct((NSK, PER, E), jnp.float32),
)


def _sc_body(meta_hbm, t_hbm, bias_hbm, out_hbm,
             meta_v, idx_v, bias_v, out_v, shared, sem, sem2):
    sid = lax.axis_index("s")
    wid = sid * 2 + lax.axis_index("c")

    stage = pltpu.async_copy(t_hbm.at[pl.ds(sid * _STAGE, _STAGE)],
                             shared.at[pl.ds(sid * _STAGE, _STAGE)], sem2)

    pltpu.sync_copy(meta_hbm.at[wid], meta_v)
    pltpu.sync_copy(bias_hbm, bias_v)

    for chunk in range(2):
        sl = pl.ds(chunk * 16, 16)
        cs = [meta_v[f, sl] for f in range(MC)]
        rs = [meta_v[MC + f, sl] for f in range(MC)]
        ps = [meta_v[2 * MC + k, sl] for k in range(MC * FL)]
        asv = [meta_v[2 * MC + MC * FL + k, sl] for k in range(MC * FL)]
        j = 0
        for f in range(MC):
            win = None
            for g in range(f + 1, MC):
                neq = cs[f] != cs[g]
                win = neq if win is None else win & neq
            base = cs[f] * PER
            seven = [base + rs[f]]
            for q in range(FL):
                seven.append(base + 25 + 25 * q + ps[FL * f + q])
            for q in range(FL):
                seven.append(base + 100 + 25 * q + asv[FL * f + q])
            for vec in seven:
                v = vec if win is None else jnp.where(win, vec, ZERO)
                idx_v[j, pl.ds(chunk * 16, 16)] = v
                j += 1

    bias_regs = [bias_v[pl.ds(k * 16, 16)] for k in range(E // 16)]

    def initb(t, carry):
        for k in range(E // 16):
            out_v[t, pl.ds(k * 16, 16)] = bias_regs[k]
        return carry

    lax.fori_loop(0, TPAD, initb, 0)

    stage.wait()
    plsc.subcore_barrier()

    copies = [
        pltpu.async_copy(shared.at[idx_v.at[j]], out_v, sem, add=True)
        for j in range(SLOTS)
    ]
    for cp in copies:
        cp.wait()
    pltpu.sync_copy(out_v.at[pl.ds(0, TPW)], out_hbm.at[pl.ds(wid * TPW, TPW)])


@functools.cache
def _sc_gather_fn():
    return functools.partial(
        pl.kernel,
        out_type=jax.ShapeDtypeStruct((NTOK, E), jnp.float32),
        mesh=plsc.VectorSubcoreMesh(
            core_axis_name="c", subcore_axis_name="s", num_cores=2, num_subcores=16),
        scratch_types=[
            pltpu.VMEM((2 * MC + 2 * MC * FL, TPAD), jnp.int32),
            pltpu.VMEM((SLOTS, TPAD), jnp.int32),
            pltpu.VMEM((E,), jnp.float32),
            pltpu.VMEM((TPAD, E), jnp.float32),
            pltpu.VMEM_SHARED((NSK * PER, E), jnp.float32),
            pltpu.SemaphoreType.DMA,
            pltpu.SemaphoreType.DMA,
        ],
        compiler_params=pltpu.CompilerParams(use_tc_tiling_on_sc=False),
    )(_sc_body)


@jax.jit
def kernel(concepts, rgaps, pcounts, acounts, emb_table_skill, W_cemb, W_pre, b_pre):
    cw = concepts.reshape(NW, TPW, MC).astype(jnp.int32).transpose(0, 2, 1)
    rw = rgaps.reshape(NW, TPW, MC).astype(jnp.int32).transpose(0, 2, 1)
    pw = pcounts.reshape(NW, TPW, MC * FL).astype(jnp.int32).transpose(0, 2, 1)
    aw = acounts.reshape(NW, TPW, MC * FL).astype(jnp.int32).transpose(0, 2, 1)
    meta = jnp.concatenate([cw, rw, pw, aw], axis=1)
    meta = jnp.pad(meta, ((0, 0), (0, 0), (0, TPAD - TPW)))

    wp4 = W_pre.reshape(E, NSK // SK_BLK, SK_BLK, E + NT)
    emb3 = emb_table_skill.reshape(NSK // SK_BLK, SK_BLK, E)
    table = _t_precompute(wp4, emb3, W_cemb)

    out = _sc_gather_fn()(meta, table.reshape(NSK * PER, E), b_pre)
    return out.reshape(B, L, E)
